# Initial kernel scaffold; baseline (speedup 1.0000x reference)
import jax
import jax.numpy as jnp
from jax.experimental import pallas as pl


def kernel(x, edge_index, params):
    raise NotImplementedError("write your pallas kernel here")


# speedup vs baseline: 2.3921x; 2.3921x over previous
import functools

import jax
import jax.numpy as jnp
from jax import lax
from jax.experimental import pallas as pl
from jax.experimental.pallas import tpu as pltpu
from jax.experimental.pallas import tpu_sc as plsc

N = 10000
D = 128
H = 128
E = 320000

NC = 2
NS = 16
NW = NC * NS
CH = 128
K = 80
T = K * CH
E_PAD = NW * T
ACC_ROWS = 10240
ZROWS = ACC_ROWS // NS
WROWS = N // NS


L = 16
GRP = CH // L
NV = H // L
DUMP = N


def _sc_segsum_body(h_hbm, src_hbm, dst_hbm, zeros_hbm, out_hbm,
                    src_v, dst_v, rows_v, agg_sp, sem):
  cid = lax.axis_index("c")
  sid = lax.axis_index("s")
  wid = cid * NS + sid

  pltpu.sync_copy(zeros_hbm.at[pl.ds(sid * ZROWS, ZROWS)],
                  agg_sp.at[pl.ds(sid * ZROWS, ZROWS)])
  pltpu.sync_copy(src_hbm.at[wid], src_v)
  pltpu.sync_copy(dst_hbm.at[wid], dst_v)
  plsc.subcore_barrier()

  def chunk(j, c):
    pltpu.async_copy(h_hbm.at[src_v.at[j]], rows_v, sem).wait()
    pltpu.sync_copy(rows_v, agg_sp.at[dst_v.at[j]], add=True)
    return c

  lax.fori_loop(0, K, chunk, 0, unroll=False)
  plsc.subcore_barrier()

  pltpu.sync_copy(agg_sp.at[pl.ds(sid * ZROWS, ZROWS)],
                  out_hbm.at[cid, pl.ds(sid * ZROWS, ZROWS)])


@functools.cache
def _get_sc_segsum():
  return pl.kernel(
      _sc_segsum_body,
      out_type=jax.ShapeDtypeStruct((NC, ACC_ROWS, H), jnp.float32),
      mesh=plsc.VectorSubcoreMesh(core_axis_name="c", subcore_axis_name="s",
                                  num_cores=NC, num_subcores=NS),
      scratch_types=[
          pltpu.VMEM((K, CH), jnp.int32),
          pltpu.VMEM((K, CH), jnp.int32),
          pltpu.VMEM((CH, H), jnp.float32),
          pltpu.VMEM_SHARED((ACC_ROWS, H), jnp.float32),
          pltpu.SemaphoreType.DMA,
      ],
  )


def _sc_segsum(h, srcr, dstr, zeros):
  return _get_sc_segsum()(h, srcr, dstr, zeros)


def _dot(a, b):
  return jnp.dot(a, b, preferred_element_type=jnp.float32)


def _mlp_body(residual, h_ref, p_ref, s_ref, w1_ref, b1_ref, g1_ref, be1_ref,
              w2_ref, b2_ref, o_ref):
  h = h_ref[...]
  z = h * s_ref[0, 0] + p_ref[0] + p_ref[1]
  z = _dot(z, w1_ref[...]) + b1_ref[...]
  m = jnp.mean(z, axis=-1, keepdims=True)
  v = jnp.mean(jnp.square(z - m), axis=-1, keepdims=True)
  z = (z - m) / jnp.sqrt(v + 1e-5) * g1_ref[...] + be1_ref[...]
  z = jnp.maximum(z, 0.0)
  z = _dot(z, w2_ref[...]) + b2_ref[...]
  z = jnp.maximum(z, 0.0)
  if residual:
    z = z + h
  o_ref[...] = z


_MLP_BR = 1000


def _tc_mlp(h, partials, p, residual):
  full = lambda s: pl.BlockSpec(s, lambda i: (0,) * len(s))
  grid = N // _MLP_BR
  body = functools.partial(_mlp_body, residual)
  return pl.pallas_call(
      body,
      grid=(grid,),
      in_specs=[
          pl.BlockSpec((_MLP_BR, H), lambda i: (i, 0)),
          pl.BlockSpec((NC, _MLP_BR, H), lambda i: (0, i, 0)),
          pl.BlockSpec(memory_space=pltpu.SMEM),
          full((H, H)), full((1, H)), full((1, H)), full((1, H)),
          full((H, H)), full((1, H)),
      ],
      out_specs=pl.BlockSpec((_MLP_BR, H), lambda i: (i, 0)),
      out_shape=jax.ShapeDtypeStruct((N, H), jnp.float32),
  )(h, partials, (1.0 + p['eps']).reshape(1, 1),
    p['W1'], p['b1'].reshape(1, H), p['g1'].reshape(1, H),
    p['be1'].reshape(1, H), p['W2'], p['b2'].reshape(1, H))


def _ln_row(z, g, b):
  m = jnp.mean(z, axis=-1, keepdims=True)
  v = jnp.mean(jnp.square(z - m), axis=-1, keepdims=True)
  return (z - m) / jnp.sqrt(v + 1e-5) * g + b


def _head_body(h_ref, wa_ref, ba_ref, a1_ref, ab1_ref, ag1_ref, abe1_ref,
               a2_ref, ab2_ref, ag2_ref, abe2_ref, a3_ref, ab3_ref,
               r1_ref, rb1_ref, rg1_ref, rbe1_ref, r2_ref, rb2_ref,
               o_ref, s_ref):
  h = h_ref[...]
  att = jax.nn.sigmoid(
      _dot(h, wa_ref[...])
      + ba_ref[0, 0])
  hw = h * att
  a = _dot(hw, a1_ref[...]) + ab1_ref[...]
  a = jnp.maximum(_ln_row(a, ag1_ref[...], abe1_ref[...]), 0.0)
  a = _dot(a, a2_ref[...]) + ab2_ref[...]
  a = jnp.maximum(_ln_row(a, ag2_ref[...], abe2_ref[...]), 0.0)
  ang = jnp.tanh(
      _dot(a, a3_ref[...])
      + ab3_ref[0, 0]) * jnp.pi
  r = _dot(hw, r1_ref[...]) + rb1_ref[...]
  r = jnp.maximum(_ln_row(r, rg1_ref[...], rbe1_ref[...]), 0.0)
  rad = 0.9 + 0.2 * jax.nn.sigmoid(
      _dot(r, r2_ref[...])
      + rb2_ref[0, 0])
  coords = jnp.concatenate([rad * jnp.cos(ang), rad * jnp.sin(ang)], axis=1)
  o_ref[...] = coords
  s_ref[...] = jnp.concatenate(
      [jnp.sum(coords, axis=0, keepdims=True), jnp.zeros((7, 2), jnp.float32)])


def _center_body(c_ref, s_ref, o_ref):
  o_ref[...] = c_ref[...] - jnp.sum(s_ref[...], axis=0, keepdims=True) / N


_HEAD_BR = 1000


def _tc_head(h, p):
  full = lambda s: pl.BlockSpec(s, lambda i: (0,) * len(s))
  grid = N // _HEAD_BR
  args = (h, p['Wa'], p['ba'].reshape(1, 1),
          p['A1'], p['ab1'].reshape(1, 2 * H), p['ag1'].reshape(1, 2 * H),
          p['abe1'].reshape(1, 2 * H),
          p['A2'], p['ab2'].reshape(1, H), p['ag2'].reshape(1, H),
          p['abe2'].reshape(1, H),
          p['A3'], p['ab3'].reshape(1, 1),
          p['R1'], p['rb1'].reshape(1, H), p['rg1'].reshape(1, H),
          p['rbe1'].reshape(1, H),
          p['R2'], p['rb2'].reshape(1, 1))
  coords, sums = pl.pallas_call(
      _head_body,
      grid=(grid,),
      in_specs=[
          pl.BlockSpec((_HEAD_BR, H), lambda i: (i, 0)),
          full((H, 1)), full((1, 1)),
          full((H, 2 * H)), full((1, 2 * H)), full((1, 2 * H)),
          full((1, 2 * H)),
          full((2 * H, H)), full((1, H)), full((1, H)), full((1, H)),
          full((H, 1)), full((1, 1)),
          full((H, H)), full((1, H)), full((1, H)), full((1, H)),
          full((H, 1)), full((1, 1)),
      ],
      out_specs=[pl.BlockSpec((_HEAD_BR, 2), lambda i: (i, 0)),
                 pl.BlockSpec((8, 2), lambda i: (i, 0))],
      out_shape=[jax.ShapeDtypeStruct((N, 2), jnp.float32),
                 jax.ShapeDtypeStruct((grid * 8, 2), jnp.float32)],
  )(*args)
  return pl.pallas_call(
      _center_body,
      grid=(grid,),
      in_specs=[pl.BlockSpec((_HEAD_BR, 2), lambda i: (i, 0)),
                full((grid * 8, 2))],
      out_specs=pl.BlockSpec((_HEAD_BR, 2), lambda i: (i, 0)),
      out_shape=jax.ShapeDtypeStruct((N, 2), jnp.float32),
  )(coords, sums)


def kernel(x, edge_index, params):
  pad = E_PAD - E
  order = jnp.argsort(edge_index[1], stable=True)
  src = jnp.concatenate([edge_index[0][order], jnp.zeros((pad,), jnp.int32)])
  dst = jnp.concatenate([edge_index[1][order], jnp.full((pad,), N, jnp.int32)])
  srcr = src.reshape(NW, K, CH)
  dstr = dst.reshape(NW, K, CH)
  zeros = jnp.zeros((ACC_ROWS, H), jnp.float32)

  h = x
  for i, name in enumerate(('c1', 'c2', 'c3', 'c4')):
    partials = _sc_segsum(h, srcr, dstr, zeros)
    h = _tc_mlp(h, partials, params[name], residual=(i > 0))
  return _tc_head(h, params)
